# baseline (speedup 1.0000x reference)
import jax
import jax.numpy as jnp
from jax import lax
from jax.experimental import pallas as pl
from jax.experimental.pallas import tpu as pltpu, tpu_sc as plsc

VOCAB = 100000
EMBED_DIM = 64
BATCH = 16384
SEQ = 200
N_TOKENS = 20
SPLIT1 = 10

NC, NS, L = 2, 16, 16
NW = NC * NS
BB = 128
NBB = BATCH // BB
BB_PER_W = NBB // NW
NG = SEQ - N_TOKENS
DO = EMBED_DIM // 8


def _body(tokT_hbm, table_hbm, learnedB_hbm, out_hbm,
          tokT_v, rows0, rows1, slab0, slab1, sg0, sg1, so0, so1, sl):
    wid = lax.axis_index("s") * NC + lax.axis_index("c")
    rows = (rows0, rows1)
    slabs = (slab0, slab1)
    sg = (sg0, sg1)
    so = (so0, so1)

    def lbody(r, carry):
        pltpu.sync_copy(learnedB_hbm.at[r], slab0)
        j = jnp.where(r < SPLIT1, r + 1, r + 2)
        for k in range(BB_PER_W):
            pltpu.async_copy(slab0, out_hbm.at[j, :, wid * BB_PER_W + k], sl)
        for k in range(BB_PER_W):
            pltpu.make_async_copy(slab0, out_hbm.at[0, :, 0], sl).wait()
        return carry

    lax.fori_loop(0, N_TOKENS, lbody, 0)

    def fire_gather(m, p):
        pltpu.async_copy(table_hbm.at[tokT_v.at[m]], rows[p], sg[p])

    def wait_gather(p):
        pltpu.make_async_copy(table_hbm.at[tokT_v.at[0]], rows[p], sg[p]).wait()

    def transpose(p, q):
        iota = lax.iota(jnp.int32, L)

        def kbody(kk, carry):
            for s in range(8):
                rot = (iota + (kk * 8 + s)) & (L - 1)
                dsub = rot & 7
                for d0 in range(0, EMBED_DIM, L):
                    col = rot + d0
                    doct = col >> 3
                    for bb0 in range(0, BB, L):
                        brow = iota + bb0
                        v = plsc.load_gather(rows[p], [brow, col])
                        plsc.store_scatter(slabs[q], [doct, dsub, brow], v)
            return carry

        lax.fori_loop(0, 2, kbody, 0)

    def fire_out(j, bblk, q):
        pltpu.async_copy(slabs[q], out_hbm.at[j, :, bblk], so[q])

    def wait_out(q):
        pltpu.make_async_copy(slabs[q], out_hbm.at[0, :, 0], so[q]).wait()

    def step(t, p):
        u = lax.rem(t, NG)
        bblk = wid * BB_PER_W + t // NG

        @pl.when(u == 0)
        def _():
            pltpu.sync_copy(tokT_hbm.at[:, pl.ds(bblk * BB, BB)], tokT_v)
            fire_gather(0, p)

        wait_gather(p)

        @pl.when(u < NG - 1)
        def _():
            fire_gather(jnp.where(u == 0, N_TOKENS + 1, u + 21), 1 - p)

        @pl.when(t >= 2)
        def _():
            wait_out(p)

        transpose(p, p)
        j = jnp.where(u == 0, 0, jnp.where(u == 1, SPLIT1 + 1, u + 20))
        fire_out(j, bblk, p)

    def gbody(s, carry):
        step(2 * s, 0)
        step(2 * s + 1, 1)
        return carry

    lax.fori_loop(0, BB_PER_W * NG // 2, gbody, 0)
    wait_out(0)
    wait_out(1)


def kernel(tokens, wte_weight, learned_embedding):
    tokT = tokens.T.astype(jnp.int32)
    learnedB = jnp.broadcast_to(
        learned_embedding.reshape(N_TOKENS, DO, 8, 1), (N_TOKENS, DO, 8, BB))
    mesh = plsc.VectorSubcoreMesh(core_axis_name="c", subcore_axis_name="s",
                                  num_cores=NC, num_subcores=NS)
    out5 = pl.kernel(
        _body,
        out_type=jax.ShapeDtypeStruct((SEQ, DO, NBB, 8, BB), jnp.float32),
        mesh=mesh,
        compiler_params=pltpu.CompilerParams(
            use_tc_tiling_on_sc=False, needs_layout_passes=False),
        scratch_types=[
            pltpu.VMEM((SEQ, BB), jnp.int32),
            pltpu.VMEM((BB, EMBED_DIM), jnp.float32),
            pltpu.VMEM((BB, EMBED_DIM), jnp.float32),
            pltpu.VMEM((DO, 8, BB), jnp.float32),
            pltpu.VMEM((DO, 8, BB), jnp.float32),
            pltpu.SemaphoreType.DMA,
            pltpu.SemaphoreType.DMA,
            pltpu.SemaphoreType.DMA,
            pltpu.SemaphoreType.DMA,
            pltpu.SemaphoreType.DMA,
        ],
    )(tokT, wte_weight, learnedB)
    t = jnp.transpose(out5, (2, 4, 0, 1, 3))
    return t.reshape(BATCH, SEQ, EMBED_DIM)


# speedup vs baseline: 1.0877x; 1.0877x over previous
import jax
import jax.numpy as jnp
from jax import lax
from jax.experimental import pallas as pl
from jax.experimental.pallas import tpu as pltpu, tpu_sc as plsc

VOCAB = 100000
EMBED_DIM = 64
BATCH = 16384
SEQ = 200
N_TOKENS = 20
SPLIT1 = 10

NC, NS, L = 2, 16, 16
NW = NC * NS
BB = 128
NBB = BATCH // BB
BB_PER_W = NBB // NW
NG = SEQ - N_TOKENS
DO = EMBED_DIM // 8


def _body(tokT_hbm, table_hbm, learnedB_hbm, out_hbm,
          tokT_v, rows0, rows1, slab0, slab1, sg0, sg1, so0, so1, sl):
    wid = lax.axis_index("s") * NC + lax.axis_index("c")
    rows = (rows0, rows1)
    slabs = (slab0, slab1)
    sg = (sg0, sg1)
    so = (so0, so1)

    def lbody(r, carry):
        pltpu.sync_copy(learnedB_hbm.at[r], slab0)
        j = jnp.where(r < SPLIT1, r + 1, r + 2)
        for k in range(BB_PER_W):
            pltpu.async_copy(slab0, out_hbm.at[j, :, wid * BB_PER_W + k], sl)
        for k in range(BB_PER_W):
            pltpu.make_async_copy(slab0, out_hbm.at[0, :, 0], sl).wait()
        return carry

    lax.fori_loop(0, N_TOKENS, lbody, 0)

    def fire_gather(m, p):
        pltpu.async_copy(table_hbm.at[tokT_v.at[m]], rows[p], sg[p])

    def wait_gather(p):
        pltpu.make_async_copy(table_hbm.at[tokT_v.at[0]], rows[p], sg[p]).wait()

    def transpose(p, q):
        iota = lax.iota(jnp.int32, L)

        def kbody(kk, carry):
            for s in range(2):
                rot = (iota + (kk * 2 + s)) & (L - 1)
                dsub = rot & 7
                for d0 in range(0, EMBED_DIM, L):
                    col = rot + d0
                    doct = col >> 3
                    for bb0 in range(0, BB, L):
                        brow = iota + bb0
                        v = plsc.load_gather(rows[p], [brow, col])
                        plsc.store_scatter(slabs[q], [doct, dsub, brow], v)
            return carry

        lax.fori_loop(0, 8, kbody, 0)

    def fire_out(j, bblk, q):
        pltpu.async_copy(slabs[q], out_hbm.at[j, :, bblk], so[q])

    def wait_out(q):
        pltpu.make_async_copy(slabs[q], out_hbm.at[0, :, 0], so[q]).wait()

    def step(t, p):
        u = lax.rem(t, NG)
        bblk = wid * BB_PER_W + t // NG

        @pl.when(u == 0)
        def _():
            pltpu.sync_copy(tokT_hbm.at[:, pl.ds(bblk * BB, BB)], tokT_v)
            fire_gather(0, p)

        wait_gather(p)

        @pl.when(u < NG - 1)
        def _():
            fire_gather(jnp.where(u == 0, N_TOKENS + 1, u + 21), 1 - p)

        @pl.when(t >= 2)
        def _():
            wait_out(p)

        transpose(p, p)
        j = jnp.where(u == 0, 0, jnp.where(u == 1, SPLIT1 + 1, u + 20))
        fire_out(j, bblk, p)

    def gbody(s, carry):
        step(2 * s, 0)
        step(2 * s + 1, 1)
        return carry

    lax.fori_loop(0, BB_PER_W * NG // 2, gbody, 0)
    wait_out(0)
    wait_out(1)


def kernel(tokens, wte_weight, learned_embedding):
    tokT = tokens.T.astype(jnp.int32)
    learnedB = jnp.broadcast_to(
        learned_embedding.reshape(N_TOKENS, DO, 8, 1), (N_TOKENS, DO, 8, BB))
    mesh = plsc.VectorSubcoreMesh(core_axis_name="c", subcore_axis_name="s",
                                  num_cores=NC, num_subcores=NS)
    out5 = pl.kernel(
        _body,
        out_type=jax.ShapeDtypeStruct((SEQ, DO, NBB, 8, BB), jnp.float32),
        mesh=mesh,
        compiler_params=pltpu.CompilerParams(
            use_tc_tiling_on_sc=False, needs_layout_passes=False),
        scratch_types=[
            pltpu.VMEM((SEQ, BB), jnp.int32),
            pltpu.VMEM((BB, EMBED_DIM), jnp.float32),
            pltpu.VMEM((BB, EMBED_DIM), jnp.float32),
            pltpu.VMEM((DO, 8, BB), jnp.float32),
            pltpu.VMEM((DO, 8, BB), jnp.float32),
            pltpu.SemaphoreType.DMA,
            pltpu.SemaphoreType.DMA,
            pltpu.SemaphoreType.DMA,
            pltpu.SemaphoreType.DMA,
            pltpu.SemaphoreType.DMA,
        ],
    )(tokT, wte_weight, learnedB)
    t = jnp.transpose(out5, (2, 4, 0, 1, 3))
    return t.reshape(BATCH, SEQ, EMBED_DIM)


# speedup vs baseline: 1.7860x; 1.6420x over previous
import jax
import jax.numpy as jnp
from jax import lax
from jax.experimental import pallas as pl
from jax.experimental.pallas import tpu as pltpu, tpu_sc as plsc

VOCAB = 100000
EMBED_DIM = 64
BATCH = 16384
SEQ = 200
N_TOKENS = 20
SPLIT1 = 10

NC, NS, L = 2, 16, 16
NW = NC * NS
BB = 128
NBB = BATCH // BB
BB_PER_W = NBB // NW
NG = SEQ - N_TOKENS
DO = EMBED_DIM // 8


def _body(tokT_hbm, table_hbm, learnedB_hbm, out_hbm,
          tokT_v, rows0, rows1, slab0, slab1, sg0, sg1, so0, so1, sl):
    wid = lax.axis_index("s") * NC + lax.axis_index("c")
    rows = (rows0, rows1)
    slabs = (slab0, slab1)
    sg = (sg0, sg1)
    so = (so0, so1)

    def lbody(r, carry):
        pltpu.sync_copy(learnedB_hbm.at[r], slab0)
        j = jnp.where(r < SPLIT1, r + 1, r + 2)
        for k in range(BB_PER_W):
            pltpu.async_copy(slab0, out_hbm.at[j, :, wid * BB_PER_W + k], sl)
        for k in range(BB_PER_W):
            pltpu.make_async_copy(slab0, out_hbm.at[0, :, 0], sl).wait()
        return carry

    lax.fori_loop(0, N_TOKENS, lbody, 0)

    def fire_gather(m, p):
        pltpu.async_copy(table_hbm.at[tokT_v.at[m]], rows[p], sg[p])

    def wait_gather(p):
        pltpu.make_async_copy(table_hbm.at[tokT_v.at[0]], rows[p], sg[p]).wait()

    def transpose(p, q):
        iota = lax.iota(jnp.int32, L)

        def kbody(kk, carry):
            for s in range(4):
                rot = (iota + (kk * 4 + s)) & (L - 1)
                dsub = rot & 7
                for d0 in range(0, EMBED_DIM, L):
                    col = rot + d0
                    doct = col >> 3
                    vs = [plsc.load_gather(rows[p], [iota + bb0, col])
                          for bb0 in range(0, BB, L)]
                    for bb0, v in zip(range(0, BB, L), vs):
                        plsc.store_scatter(slabs[q], [doct, dsub, iota + bb0], v)
            return carry

        lax.fori_loop(0, 4, kbody, 0)

    def fire_out(j, bblk, q):
        pltpu.async_copy(slabs[q], out_hbm.at[j, :, bblk], so[q])

    def wait_out(q):
        pltpu.make_async_copy(slabs[q], out_hbm.at[0, :, 0], so[q]).wait()

    def step(t, p):
        u = lax.rem(t, NG)
        bblk = wid * BB_PER_W + t // NG

        @pl.when(u == 0)
        def _():
            pltpu.sync_copy(tokT_hbm.at[:, pl.ds(bblk * BB, BB)], tokT_v)
            fire_gather(0, p)

        wait_gather(p)

        @pl.when(u < NG - 1)
        def _():
            fire_gather(jnp.where(u == 0, N_TOKENS + 1, u + 21), 1 - p)

        @pl.when(t >= 2)
        def _():
            wait_out(p)

        transpose(p, p)
        j = jnp.where(u == 0, 0, jnp.where(u == 1, SPLIT1 + 1, u + 20))
        fire_out(j, bblk, p)

    def gbody(s, carry):
        step(2 * s, 0)
        step(2 * s + 1, 1)
        return carry

    lax.fori_loop(0, BB_PER_W * NG // 2, gbody, 0)
    wait_out(0)
    wait_out(1)


def kernel(tokens, wte_weight, learned_embedding):
    tokT = tokens.T.astype(jnp.int32)
    learnedB = jnp.broadcast_to(
        learned_embedding.reshape(N_TOKENS, DO, 8, 1), (N_TOKENS, DO, 8, BB))
    mesh = plsc.VectorSubcoreMesh(core_axis_name="c", subcore_axis_name="s",
                                  num_cores=NC, num_subcores=NS)
    out5 = pl.kernel(
        _body,
        out_type=jax.ShapeDtypeStruct((SEQ, DO, NBB, 8, BB), jnp.float32),
        mesh=mesh,
        compiler_params=pltpu.CompilerParams(
            use_tc_tiling_on_sc=False, needs_layout_passes=False),
        scratch_types=[
            pltpu.VMEM((SEQ, BB), jnp.int32),
            pltpu.VMEM((BB, EMBED_DIM), jnp.float32),
            pltpu.VMEM((BB, EMBED_DIM), jnp.float32),
            pltpu.VMEM((DO, 8, BB), jnp.float32),
            pltpu.VMEM((DO, 8, BB), jnp.float32),
            pltpu.SemaphoreType.DMA,
            pltpu.SemaphoreType.DMA,
            pltpu.SemaphoreType.DMA,
            pltpu.SemaphoreType.DMA,
            pltpu.SemaphoreType.DMA,
        ],
    )(tokT, wte_weight, learnedB)
    t = jnp.transpose(out5, (2, 4, 0, 1, 3))
    return t.reshape(BATCH, SEQ, EMBED_DIM)


# speedup vs baseline: 1.8065x; 1.0115x over previous
import jax
import jax.numpy as jnp
from jax import lax
from jax.experimental import pallas as pl
from jax.experimental.pallas import tpu as pltpu, tpu_sc as plsc

VOCAB = 100000
EMBED_DIM = 64
BATCH = 16384
SEQ = 200
N_TOKENS = 20
SPLIT1 = 10

NC, NS, L = 2, 16, 16
NW = NC * NS
BB = 128
NBB = BATCH // BB
BB_PER_W = NBB // NW
NG = SEQ - N_TOKENS
DO = EMBED_DIM // 8


def _body(tokT_hbm, table_hbm, learnedB_hbm, out_hbm,
          tokT_v, rows0, rows1, slab0, slab1, sg0, sg1, so0, so1, sl):
    wid = lax.axis_index("s") * NC + lax.axis_index("c")
    rows = (rows0, rows1)
    slabs = (slab0, slab1)
    sg = (sg0, sg1)
    so = (so0, so1)

    def _lstage(r, p):
        pltpu.async_copy(learnedB_hbm.at[r], slabs[p], sg[p])

    def _lwrites(r, p):
        j = r + 1 if r < SPLIT1 else r + 2
        for k in range(BB_PER_W):
            pltpu.async_copy(slabs[p], out_hbm.at[j, :, wid * BB_PER_W + k],
                             so[p])

    def _lwait_writes(p):
        for _ in range(BB_PER_W):
            pltpu.make_async_copy(slabs[p], out_hbm.at[0, :, 0], so[p]).wait()

    _lstage(0, 0)
    for r in range(N_TOKENS):
        p = r % 2
        pltpu.make_async_copy(learnedB_hbm.at[0], slabs[p], sg[p]).wait()
        _lwrites(r, p)
        if r + 1 < N_TOKENS:
            if r >= 1:
                _lwait_writes(1 - p)
            _lstage(r + 1, 1 - p)
    _lwait_writes(0)
    _lwait_writes(1)

    def fire_gather(m, p):
        pltpu.async_copy(table_hbm.at[tokT_v.at[m]], rows[p], sg[p])

    def wait_gather(p):
        pltpu.make_async_copy(table_hbm.at[tokT_v.at[0]], rows[p], sg[p]).wait()

    def transpose(p, q):
        iota = lax.iota(jnp.int32, L)

        def kbody(kk, carry):
            for s in range(4):
                rot = (iota + (kk * 4 + s)) & (L - 1)
                dsub = rot & 7
                for d0 in range(0, EMBED_DIM, L):
                    col = rot + d0
                    doct = col >> 3
                    vs = [plsc.load_gather(rows[p], [iota + bb0, col])
                          for bb0 in range(0, BB, L)]
                    for bb0, v in zip(range(0, BB, L), vs):
                        plsc.store_scatter(slabs[q], [doct, dsub, iota + bb0], v)
            return carry

        lax.fori_loop(0, 4, kbody, 0)

    def fire_out(j, bblk, q):
        pltpu.async_copy(slabs[q], out_hbm.at[j, :, bblk], so[q])

    def wait_out(q):
        pltpu.make_async_copy(slabs[q], out_hbm.at[0, :, 0], so[q]).wait()

    def step(t, p):
        u = lax.rem(t, NG)
        bblk = wid * BB_PER_W + t // NG

        @pl.when(u == 0)
        def _():
            pltpu.sync_copy(tokT_hbm.at[:, pl.ds(bblk * BB, BB)], tokT_v)
            fire_gather(0, p)

        wait_gather(p)

        @pl.when(u < NG - 1)
        def _():
            fire_gather(jnp.where(u == 0, N_TOKENS + 1, u + 21), 1 - p)

        @pl.when(t >= 2)
        def _():
            wait_out(p)

        transpose(p, p)
        j = jnp.where(u == 0, 0, jnp.where(u == 1, SPLIT1 + 1, u + 20))
        fire_out(j, bblk, p)

    def gbody(s, carry):
        step(2 * s, 0)
        step(2 * s + 1, 1)
        return carry

    lax.fori_loop(0, BB_PER_W * NG // 2, gbody, 0)
    wait_out(0)
    wait_out(1)


def kernel(tokens, wte_weight, learned_embedding):
    tokT = tokens.T.astype(jnp.int32)
    learnedB = jnp.broadcast_to(
        learned_embedding.reshape(N_TOKENS, DO, 8, 1), (N_TOKENS, DO, 8, BB))
    mesh = plsc.VectorSubcoreMesh(core_axis_name="c", subcore_axis_name="s",
                                  num_cores=NC, num_subcores=NS)
    out5 = pl.kernel(
        _body,
        out_type=jax.ShapeDtypeStruct((SEQ, DO, NBB, 8, BB), jnp.float32),
        mesh=mesh,
        compiler_params=pltpu.CompilerParams(
            use_tc_tiling_on_sc=False, needs_layout_passes=False),
        scratch_types=[
            pltpu.VMEM((SEQ, BB), jnp.int32),
            pltpu.VMEM((BB, EMBED_DIM), jnp.float32),
            pltpu.VMEM((BB, EMBED_DIM), jnp.float32),
            pltpu.VMEM((DO, 8, BB), jnp.float32),
            pltpu.VMEM((DO, 8, BB), jnp.float32),
            pltpu.SemaphoreType.DMA,
            pltpu.SemaphoreType.DMA,
            pltpu.SemaphoreType.DMA,
            pltpu.SemaphoreType.DMA,
            pltpu.SemaphoreType.DMA,
        ],
    )(tokT, wte_weight, learnedB)
    t = jnp.transpose(out5, (2, 4, 0, 1, 3))
    return t.reshape(BATCH, SEQ, EMBED_DIM)
